# Initial kernel scaffold; baseline (speedup 1.0000x reference)
import jax
import jax.numpy as jnp
from jax.experimental import pallas as pl


def kernel(x, conv1_w, conv1_b, conv2_w, conv2_b, conv3_w, conv3_b, fc1_w, fc1_b, head_w, head_b):
    raise NotImplementedError("write your pallas kernel here")


# speedup vs baseline: 19.5210x; 19.5210x over previous
import jax
import jax.numpy as jnp
from jax.experimental import pallas as pl
from jax.experimental.pallas import tpu as pltpu

_N_ACT = 7


def _pad_hw(y, ph, pw):
    b, h, w, c = y.shape
    zr = jnp.zeros((b, ph, w, c), y.dtype)
    y = jnp.concatenate([zr, y, zr], axis=1)
    zc = jnp.zeros((b, h + 2 * ph, pw, c), y.dtype)
    return jnp.concatenate([zc, y, zc], axis=2)


def _tower_kernel(xs_ref, w1_ref, b1_ref, w2_ref, b2_ref, w3_ref, b3_ref,
                  o_ref):
    bb = xs_ref.shape[0]
    xs = xs_ref[...]

    a = jnp.concatenate(
        [xs[:, :16, :20, :], xs[:, 1:, :20, :],
         xs[:, :16, 1:, :], xs[:, 1:, 1:, :]], axis=-1)
    y1 = jnp.dot(a.reshape(bb * 320, 1024), w1_ref[...],
                 preferred_element_type=jnp.float32)
    y1 = jnp.maximum(y1 + b1_ref[...], 0.0).reshape(bb, 16, 20, 32)

    y1p = _pad_hw(y1, 1, 1)
    yr = y1p.reshape(bb, 9, 2, 11, 2, 32)
    taps2 = []
    for i in range(4):
        for j in range(4):
            taps2.append(yr[:, i // 2:i // 2 + 8, i % 2,
                            j // 2:j // 2 + 10, j % 2, :])
    p2 = jnp.concatenate(taps2, axis=-1)
    y2 = jnp.dot(p2.reshape(bb * 80, 512), w2_ref[...],
                 preferred_element_type=jnp.float32)
    y2 = jnp.maximum(y2 + b2_ref[...], 0.0).reshape(bb, 8, 10, 64)

    y2p = _pad_hw(y2, 1, 1)
    taps3 = [y2p[:, i:i + 8, j:j + 10, :]
             for i in range(3) for j in range(3)]
    p3 = jnp.concatenate(taps3, axis=-1)
    y3 = jnp.dot(p3.reshape(bb * 80, 576), w3_ref[...],
                 preferred_element_type=jnp.float32)
    y3 = jnp.maximum(y3 + b3_ref[...], 0.0)

    o_ref[...] = y3.reshape(1, bb * 80, 64)


def _fc_head_kernel(f_ref, w_ref, b_ref, hw_ref, hb_ref, o_ref):
    h = jnp.dot(f_ref[...], w_ref[...], preferred_element_type=jnp.float32)
    h = jnp.maximum(h + b_ref[...], 0.0)
    s = jnp.dot(h, hw_ref[...], preferred_element_type=jnp.float32)
    s = s + hb_ref[...]
    adv = s[:, :_N_ACT]
    val = s[:, _N_ACT:_N_ACT + 1]
    o_ref[...] = val + adv - jnp.mean(adv, axis=1, keepdims=True)


def kernel(x, conv1_w, conv1_b, conv2_w, conv2_b, conv3_w, conv3_b,
           fc1_w, fc1_b, head_w, head_b):
    B = x.shape[0]

    xn = jnp.transpose(x, (0, 2, 3, 1))
    xp = jnp.pad(xn, ((0, 0), (1, 7), (4, 6), (0, 0)))
    xs = xp.reshape(B, 17, 8, 21, 8, 4).transpose(0, 1, 3, 2, 4, 5)
    xs = xs.reshape(B, 17, 21, 256)

    i = jnp.arange(10)[:, None, None]
    j = jnp.arange(14)[None, :, None]
    ch = jnp.arange(4)[None, None, :]
    q = (i // 8) + 2 * (j // 8)
    idx = (q * 256 + (i % 8) * 32 + (j % 8) * 4 + ch).reshape(-1)
    w1 = jnp.zeros((1024, 32), jnp.float32).at[idx].set(
        conv1_w.reshape(560, 32))
    w2 = conv2_w.reshape(512, 64)
    w3 = conv3_w.reshape(576, 64)

    G = 8
    bb = B // G
    flat = pl.pallas_call(
        _tower_kernel,
        out_shape=jax.ShapeDtypeStruct((G, bb * 80, 64), jnp.float32),
        grid_spec=pltpu.PrefetchScalarGridSpec(
            num_scalar_prefetch=0,
            grid=(G,),
            in_specs=[
                pl.BlockSpec((bb, 17, 21, 256), lambda g: (g, 0, 0, 0)),
                pl.BlockSpec((1024, 32), lambda g: (0, 0)),
                pl.BlockSpec((1, 32), lambda g: (0, 0)),
                pl.BlockSpec((512, 64), lambda g: (0, 0)),
                pl.BlockSpec((1, 64), lambda g: (0, 0)),
                pl.BlockSpec((576, 64), lambda g: (0, 0)),
                pl.BlockSpec((1, 64), lambda g: (0, 0)),
            ],
            out_specs=pl.BlockSpec((1, bb * 80, 64), lambda g: (g, 0, 0)),
        ),
        compiler_params=pltpu.CompilerParams(
            dimension_semantics=("parallel",),
        ),
    )(xs, w1, conv1_b, w2, conv2_b, w3, conv3_b)
    flat = flat.reshape(B, 5120)

    return pl.pallas_call(
        _fc_head_kernel,
        out_shape=jax.ShapeDtypeStruct((B, _N_ACT), jnp.float32),
    )(flat, fc1_w, fc1_b, head_w, head_b)


# speedup vs baseline: 20.0948x; 1.0294x over previous
import jax
import jax.numpy as jnp
from jax.experimental import pallas as pl
from jax.experimental.pallas import tpu as pltpu

_N_ACT = 7


def _pad_hw(y, ph, pw):
    b, h, w, c = y.shape
    zr = jnp.zeros((b, ph, w, c), y.dtype)
    y = jnp.concatenate([zr, y, zr], axis=1)
    zc = jnp.zeros((b, h + 2 * ph, pw, c), y.dtype)
    return jnp.concatenate([zc, y, zc], axis=2)


def _tower_kernel(xs_ref, w1_ref, b1_ref, w2_ref, b2_ref, w3_ref, b3_ref,
                  o_ref):
    bb = xs_ref.shape[0]
    xs = xs_ref[...]

    a = jnp.concatenate(
        [xs[:, :16, :20, :], xs[:, 1:, :20, :],
         xs[:, :16, 1:, :], xs[:, 1:, 1:, :]], axis=-1)
    y1 = jnp.dot(a.reshape(bb * 320, 1024), w1_ref[...],
                 preferred_element_type=jnp.float32)
    y1 = jnp.maximum(y1 + b1_ref[...], 0.0).reshape(bb, 16, 20, 32)

    y1p = _pad_hw(y1, 1, 1)
    yr = y1p.reshape(bb, 9, 2, 11, 2, 32)
    taps2 = []
    for i in range(4):
        for j in range(4):
            taps2.append(yr[:, i // 2:i // 2 + 8, i % 2,
                            j // 2:j // 2 + 10, j % 2, :])
    p2 = jnp.concatenate(taps2, axis=-1)
    y2 = jnp.dot(p2.reshape(bb * 80, 512), w2_ref[...],
                 preferred_element_type=jnp.float32)
    y2 = jnp.maximum(y2 + b2_ref[...], 0.0).reshape(bb, 8, 10, 64)

    y2p = _pad_hw(y2, 1, 1)
    taps3 = [y2p[:, i:i + 8, j:j + 10, :]
             for i in range(3) for j in range(3)]
    p3 = jnp.concatenate(taps3, axis=-1)
    y3 = jnp.dot(p3.reshape(bb * 80, 576), w3_ref[...],
                 preferred_element_type=jnp.float32)
    y3 = jnp.maximum(y3 + b3_ref[...], 0.0)

    y3r = y3.reshape(bb, 80, 64)
    for p in range(80):
        o_ref[0, :, 64 * p:64 * p + 64] = y3r[:, p, :]


def _fc_head_kernel(f_ref, w_ref, b_ref, hw_ref, hb_ref, o_ref):
    h = jnp.dot(f_ref[...], w_ref[...], preferred_element_type=jnp.float32)
    h = jnp.maximum(h + b_ref[...], 0.0)
    s = jnp.dot(h, hw_ref[...], preferred_element_type=jnp.float32)
    s = s + hb_ref[...]
    adv = s[:, :_N_ACT]
    val = s[:, _N_ACT:_N_ACT + 1]
    o_ref[...] = val + adv - jnp.mean(adv, axis=1, keepdims=True)


def kernel(x, conv1_w, conv1_b, conv2_w, conv2_b, conv3_w, conv3_b,
           fc1_w, fc1_b, head_w, head_b):
    B = x.shape[0]

    eps = x[0, 0, 0, 0] * 0.0
    xn = jnp.transpose(x, (0, 2, 3, 1))
    xp = jnp.pad(xn, ((0, 0), (1, 7), (4, 6), (0, 0)))
    xs = xp.reshape(B, 17, 8, 21, 8, 4).transpose(0, 1, 3, 2, 4, 5)
    xs = xs.reshape(B, 17, 21, 256) + eps

    i = jnp.arange(10)[:, None, None]
    j = jnp.arange(14)[None, :, None]
    ch = jnp.arange(4)[None, None, :]
    q = (i // 8) + 2 * (j // 8)
    idx = (q * 256 + (i % 8) * 32 + (j % 8) * 4 + ch).reshape(-1)
    w1 = jnp.zeros((1024, 32), jnp.float32).at[idx].set(
        conv1_w.reshape(560, 32))
    w2 = conv2_w.reshape(512, 64)
    w3 = conv3_w.reshape(576, 64)

    G = 4
    bb = B // G
    flat = pl.pallas_call(
        _tower_kernel,
        out_shape=jax.ShapeDtypeStruct((G, bb, 5120), jnp.float32),
        grid_spec=pltpu.PrefetchScalarGridSpec(
            num_scalar_prefetch=0,
            grid=(G,),
            in_specs=[
                pl.BlockSpec((bb, 17, 21, 256), lambda g: (g, 0, 0, 0)),
                pl.BlockSpec((1024, 32), lambda g: (0, 0)),
                pl.BlockSpec((1, 32), lambda g: (0, 0)),
                pl.BlockSpec((512, 64), lambda g: (0, 0)),
                pl.BlockSpec((1, 64), lambda g: (0, 0)),
                pl.BlockSpec((576, 64), lambda g: (0, 0)),
                pl.BlockSpec((1, 64), lambda g: (0, 0)),
            ],
            out_specs=pl.BlockSpec((1, bb, 5120), lambda g: (g, 0, 0)),
        ),
        compiler_params=pltpu.CompilerParams(
            dimension_semantics=("parallel",),
        ),
    )(xs, w1, conv1_b, w2, conv2_b, w3, conv3_b)
    flat = flat.reshape(B, 5120)

    return pl.pallas_call(
        _fc_head_kernel,
        out_shape=jax.ShapeDtypeStruct((B, _N_ACT), jnp.float32),
    )(flat, fc1_w, fc1_b, head_w, head_b)


# speedup vs baseline: 20.1342x; 1.0020x over previous
import jax
import jax.numpy as jnp
from jax.experimental import pallas as pl
from jax.experimental.pallas import tpu as pltpu

_N_ACT = 7


def _pad_hw(y, ph, pw):
    b, h, w, c = y.shape
    zr = jnp.zeros((b, ph, w, c), y.dtype)
    y = jnp.concatenate([zr, y, zr], axis=1)
    zc = jnp.zeros((b, h + 2 * ph, pw, c), y.dtype)
    return jnp.concatenate([zc, y, zc], axis=2)


def _tower_kernel(xs_ref, w1_ref, b1_ref, w2_ref, b2_ref, w3_ref, b3_ref,
                  o_ref):
    bb = xs_ref.shape[0]
    xs = xs_ref[...]

    a = jnp.concatenate(
        [xs[:, :16, :20, :], xs[:, 1:, :20, :],
         xs[:, :16, 1:, :], xs[:, 1:, 1:, :]], axis=-1)
    y1 = jnp.dot(a.reshape(bb * 320, 1024), w1_ref[...],
                 preferred_element_type=jnp.float32)
    y1 = jnp.maximum(y1 + b1_ref[...], 0.0).reshape(bb, 16, 20, 32)

    y1p = _pad_hw(y1, 1, 1)
    yr = y1p.reshape(bb, 9, 2, 11, 2, 32)
    taps2 = []
    for i in range(4):
        for j in range(4):
            taps2.append(yr[:, i // 2:i // 2 + 8, i % 2,
                            j // 2:j // 2 + 10, j % 2, :])
    p2 = jnp.concatenate(taps2, axis=-1)
    y2 = jnp.dot(p2.reshape(bb * 80, 512), w2_ref[...],
                 preferred_element_type=jnp.float32)
    y2 = jnp.maximum(y2 + b2_ref[...], 0.0).reshape(bb, 8, 10, 64)

    y2p = _pad_hw(y2, 1, 1)
    taps3 = [y2p[:, i:i + 8, j:j + 10, :]
             for i in range(3) for j in range(3)]
    p3 = jnp.concatenate(taps3, axis=-1)
    y3 = jnp.dot(p3.reshape(bb * 80, 576), w3_ref[...],
                 preferred_element_type=jnp.float32)
    y3 = jnp.maximum(y3 + b3_ref[...], 0.0)

    y3r = y3.reshape(bb, 80, 64)
    for p in range(80):
        o_ref[0, :, 64 * p:64 * p + 64] = y3r[:, p, :]


def _fc_head_kernel(f_ref, w_ref, b_ref, hw_ref, hb_ref, o_ref):
    h = jnp.dot(f_ref[...], w_ref[...], preferred_element_type=jnp.float32)
    h = jnp.maximum(h + b_ref[...], 0.0)
    s = jnp.dot(h, hw_ref[...], preferred_element_type=jnp.float32)
    s = s + hb_ref[...]
    adv = s[:, :_N_ACT]
    val = s[:, _N_ACT:_N_ACT + 1]
    o_ref[...] = val + adv - jnp.mean(adv, axis=1, keepdims=True)


def kernel(x, conv1_w, conv1_b, conv2_w, conv2_b, conv3_w, conv3_b,
           fc1_w, fc1_b, head_w, head_b):
    B = x.shape[0]

    xp = jnp.pad(x, ((0, 0), (0, 0), (1, 7), (4, 6)))
    xs = xp.reshape(B, 4, 17, 8, 21, 8).transpose(0, 2, 4, 3, 5, 1)
    xs = xs.reshape(B, 17, 21, 256)

    i = jnp.arange(10)[:, None, None]
    j = jnp.arange(14)[None, :, None]
    ch = jnp.arange(4)[None, None, :]
    q = (i // 8) + 2 * (j // 8)
    idx = (q * 256 + (i % 8) * 32 + (j % 8) * 4 + ch).reshape(-1)
    w1 = jnp.zeros((1024, 32), jnp.float32).at[idx].set(
        conv1_w.reshape(560, 32))
    w2 = conv2_w.reshape(512, 64)
    w3 = conv3_w.reshape(576, 64)

    G = 4
    bb = B // G
    flat = pl.pallas_call(
        _tower_kernel,
        out_shape=jax.ShapeDtypeStruct((G, bb, 5120), jnp.float32),
        grid_spec=pltpu.PrefetchScalarGridSpec(
            num_scalar_prefetch=0,
            grid=(G,),
            in_specs=[
                pl.BlockSpec((bb, 17, 21, 256), lambda g: (g, 0, 0, 0)),
                pl.BlockSpec((1024, 32), lambda g: (0, 0)),
                pl.BlockSpec((1, 32), lambda g: (0, 0)),
                pl.BlockSpec((512, 64), lambda g: (0, 0)),
                pl.BlockSpec((1, 64), lambda g: (0, 0)),
                pl.BlockSpec((576, 64), lambda g: (0, 0)),
                pl.BlockSpec((1, 64), lambda g: (0, 0)),
            ],
            out_specs=pl.BlockSpec((1, bb, 5120), lambda g: (g, 0, 0)),
        ),
        compiler_params=pltpu.CompilerParams(
            dimension_semantics=("parallel",),
        ),
    )(xs, w1, conv1_b, w2, conv2_b, w3, conv3_b)
    flat = flat.reshape(B, 5120)

    return pl.pallas_call(
        _fc_head_kernel,
        out_shape=jax.ShapeDtypeStruct((B, _N_ACT), jnp.float32),
    )(flat, fc1_w, fc1_b, head_w, head_b)


# speedup vs baseline: 23.7195x; 1.1781x over previous
import jax
import jax.numpy as jnp
from jax.experimental import pallas as pl
from jax.experimental.pallas import tpu as pltpu

_N_ACT = 7


def _pad_hw(y, ph, pw):
    b, h, w, c = y.shape
    zr = jnp.zeros((b, ph, w, c), y.dtype)
    y = jnp.concatenate([zr, y, zr], axis=1)
    zc = jnp.zeros((b, h + 2 * ph, pw, c), y.dtype)
    return jnp.concatenate([zc, y, zc], axis=2)


def _tower_kernel(xt_ref, w1_ref, b1_ref, w2_ref, b2_ref, w3_ref, b3_ref,
                  o_ref):
    bb = xt_ref.shape[0]
    xt = xt_ref[...]

    z = jnp.zeros((bb, 16, 2, 64), xt.dtype)
    z2 = jnp.zeros((bb, 16, 3, 64), xt.dtype)
    xp1 = jnp.concatenate([z, xt, z2], axis=2)
    zh = jnp.zeros((bb, 1, 84, 64), xt.dtype)
    xp1 = jnp.concatenate([zh, xp1, zh], axis=1)
    xp1 = xp1.reshape(bb, 18, 21, 4, 64)
    taps1 = []
    for ash in range(3):
        for e in range(7):
            taps1.append(xp1[:, ash:ash + 16, e // 4:e // 4 + 20, e % 4, :])
    a = jnp.concatenate(taps1, axis=-1)
    y1 = jnp.dot(a.reshape(bb * 320, 1344), w1_ref[...],
                 preferred_element_type=jnp.float32)
    y1 = jnp.maximum(y1 + b1_ref[...], 0.0).reshape(bb, 16, 20, 32)

    y1p = _pad_hw(y1, 1, 1)
    yr = y1p.reshape(bb, 9, 2, 11, 2, 32)
    taps2 = []
    for i in range(4):
        for j in range(4):
            taps2.append(yr[:, i // 2:i // 2 + 8, i % 2,
                            j // 2:j // 2 + 10, j % 2, :])
    p2 = jnp.concatenate(taps2, axis=-1)
    y2 = jnp.dot(p2.reshape(bb * 80, 512), w2_ref[...],
                 preferred_element_type=jnp.float32)
    y2 = jnp.maximum(y2 + b2_ref[...], 0.0).reshape(bb, 8, 10, 64)

    y2p = _pad_hw(y2, 1, 1)
    taps3 = [y2p[:, i:i + 8, j:j + 10, :]
             for i in range(3) for j in range(3)]
    p3 = jnp.concatenate(taps3, axis=-1)
    y3 = jnp.dot(p3.reshape(bb * 80, 576), w3_ref[...],
                 preferred_element_type=jnp.float32)
    y3 = jnp.maximum(y3 + b3_ref[...], 0.0)

    y3r = y3.reshape(bb, 80, 64)
    for p in range(80):
        o_ref[0, :, 64 * p:64 * p + 64] = y3r[:, p, :]


def _fc_head_kernel(f_ref, w_ref, b_ref, hw_ref, hb_ref, o_ref):
    h = jnp.dot(f_ref[...], w_ref[...], preferred_element_type=jnp.float32)
    h = jnp.maximum(h + b_ref[...], 0.0)
    s = jnp.dot(h, hw_ref[...], preferred_element_type=jnp.float32)
    s = s + hb_ref[...]
    adv = s[:, :_N_ACT]
    val = s[:, _N_ACT:_N_ACT + 1]
    o_ref[...] = val + adv - jnp.mean(adv, axis=1, keepdims=True)


def kernel(x, conv1_w, conv1_b, conv2_w, conv2_b, conv3_w, conv3_b,
           fc1_w, fc1_b, head_w, head_b):
    B = x.shape[0]

    xt = x.reshape(B, 4, 16, 8, 79, 2).transpose(0, 2, 4, 3, 5, 1)
    xt = xt.reshape(B, 16, 79, 64)

    i = jnp.arange(10)[:, None, None]
    j = jnp.arange(14)[None, :, None]
    ch = jnp.arange(4)[None, None, :]
    ash = (i + 7) // 8
    ri = (i + 7) % 8
    e = (j - 4) // 2 + 2
    wi2 = (j - 4) % 2
    idx = ((ash * 7 + e) * 64 + ri * 8 + wi2 * 4 + ch).reshape(-1)
    sel = (jnp.arange(1344)[:, None] == idx[None, :]).astype(jnp.float32)
    w1 = jnp.dot(sel, conv1_w.reshape(560, 32))
    w2 = conv2_w.reshape(512, 64)
    w3 = conv3_w.reshape(576, 64)

    G = 4
    bb = B // G
    flat = pl.pallas_call(
        _tower_kernel,
        out_shape=jax.ShapeDtypeStruct((G, bb, 5120), jnp.float32),
        grid_spec=pltpu.PrefetchScalarGridSpec(
            num_scalar_prefetch=0,
            grid=(G,),
            in_specs=[
                pl.BlockSpec((bb, 16, 79, 64), lambda g: (g, 0, 0, 0)),
                pl.BlockSpec((1344, 32), lambda g: (0, 0)),
                pl.BlockSpec((1, 32), lambda g: (0, 0)),
                pl.BlockSpec((512, 64), lambda g: (0, 0)),
                pl.BlockSpec((1, 64), lambda g: (0, 0)),
                pl.BlockSpec((576, 64), lambda g: (0, 0)),
                pl.BlockSpec((1, 64), lambda g: (0, 0)),
            ],
            out_specs=pl.BlockSpec((1, bb, 5120), lambda g: (g, 0, 0)),
        ),
        compiler_params=pltpu.CompilerParams(
            dimension_semantics=("parallel",),
        ),
    )(xt, w1, conv1_b, w2, conv2_b, w3, conv3_b)
    flat = flat.reshape(B, 5120)

    return pl.pallas_call(
        _fc_head_kernel,
        out_shape=jax.ShapeDtypeStruct((B, _N_ACT), jnp.float32),
    )(flat, fc1_w, fc1_b, head_w, head_b)


# speedup vs baseline: 24.7631x; 1.0440x over previous
import jax
import jax.numpy as jnp
from jax.experimental import pallas as pl
from jax.experimental.pallas import tpu as pltpu

_N_ACT = 7


def _pad_hw(y, ph, pw):
    b, h, w, c = y.shape
    zr = jnp.zeros((b, ph, w, c), y.dtype)
    y = jnp.concatenate([zr, y, zr], axis=1)
    zc = jnp.zeros((b, h + 2 * ph, pw, c), y.dtype)
    return jnp.concatenate([zc, y, zc], axis=2)


def _tower_kernel(xt_ref, w1_ref, b1_ref, w2_ref, b2_ref, w3_ref, b3_ref,
                  o_ref):
    bb = xt_ref.shape[0]
    xt = xt_ref[...].astype(jnp.float32)

    z = jnp.zeros((bb, 16, 2, 64), xt.dtype)
    z2 = jnp.zeros((bb, 16, 3, 64), xt.dtype)
    xp1 = jnp.concatenate([z, xt, z2], axis=2)
    zh = jnp.zeros((bb, 1, 84, 64), xt.dtype)
    xp1 = jnp.concatenate([zh, xp1, zh], axis=1)
    xp1 = xp1.reshape(bb, 18, 21, 4, 64)
    taps1 = []
    for ash in range(3):
        for e in range(7):
            taps1.append(xp1[:, ash:ash + 16, e // 4:e // 4 + 20, e % 4, :])
    a = jnp.concatenate(taps1, axis=-1)
    y1 = jnp.dot(a.reshape(bb * 320, 1344), w1_ref[...],
                 preferred_element_type=jnp.float32)
    y1 = jnp.maximum(y1 + b1_ref[...], 0.0).reshape(bb, 16, 20, 32)

    y1p = _pad_hw(y1, 1, 1)
    yr = y1p.reshape(bb, 9, 2, 11, 2, 32)
    taps2 = []
    for i in range(4):
        for j in range(4):
            taps2.append(yr[:, i // 2:i // 2 + 8, i % 2,
                            j // 2:j // 2 + 10, j % 2, :])
    p2 = jnp.concatenate(taps2, axis=-1)
    y2 = jnp.dot(p2.reshape(bb * 80, 512), w2_ref[...],
                 preferred_element_type=jnp.float32)
    y2 = jnp.maximum(y2 + b2_ref[...], 0.0).reshape(bb, 8, 10, 64)

    y2p = _pad_hw(y2, 1, 1)
    taps3 = [y2p[:, i:i + 8, j:j + 10, :]
             for i in range(3) for j in range(3)]
    p3 = jnp.concatenate(taps3, axis=-1)
    y3 = jnp.dot(p3.reshape(bb * 80, 576), w3_ref[...],
                 preferred_element_type=jnp.float32)
    y3 = jnp.maximum(y3 + b3_ref[...], 0.0)

    y3r = y3.reshape(bb, 80, 64)
    for p in range(80):
        o_ref[0, :, 64 * p:64 * p + 64] = y3r[:, p, :]


def _fc_head_kernel(f_ref, w_ref, b_ref, hw_ref, hb_ref, o_ref):
    h = jnp.dot(f_ref[...], w_ref[...], preferred_element_type=jnp.float32)
    h = jnp.maximum(h + b_ref[...], 0.0)
    s = jnp.dot(h, hw_ref[...], preferred_element_type=jnp.float32)
    s = s + hb_ref[...]
    adv = s[:, :_N_ACT]
    val = s[:, _N_ACT:_N_ACT + 1]
    o_ref[...] = val + adv - jnp.mean(adv, axis=1, keepdims=True)


def kernel(x, conv1_w, conv1_b, conv2_w, conv2_b, conv3_w, conv3_b,
           fc1_w, fc1_b, head_w, head_b):
    B = x.shape[0]

    xt = x.astype(jnp.bfloat16)
    xt = xt.reshape(B, 4, 16, 8, 79, 2).transpose(0, 2, 4, 3, 5, 1)
    xt = xt.reshape(B, 16, 79, 64)

    i = jnp.arange(10)[:, None, None]
    j = jnp.arange(14)[None, :, None]
    ch = jnp.arange(4)[None, None, :]
    ash = (i + 7) // 8
    ri = (i + 7) % 8
    e = (j - 4) // 2 + 2
    wi2 = (j - 4) % 2
    idx = ((ash * 7 + e) * 64 + ri * 8 + wi2 * 4 + ch).reshape(-1)
    sel = (jnp.arange(1344)[:, None] == idx[None, :]).astype(jnp.float32)
    w1 = jnp.dot(sel, conv1_w.reshape(560, 32))
    w2 = conv2_w.reshape(512, 64)
    w3 = conv3_w.reshape(576, 64)

    G = 4
    bb = B // G
    flat = pl.pallas_call(
        _tower_kernel,
        out_shape=jax.ShapeDtypeStruct((G, bb, 5120), jnp.float32),
        grid_spec=pltpu.PrefetchScalarGridSpec(
            num_scalar_prefetch=0,
            grid=(G,),
            in_specs=[
                pl.BlockSpec((bb, 16, 79, 64), lambda g: (g, 0, 0, 0)),
                pl.BlockSpec((1344, 32), lambda g: (0, 0)),
                pl.BlockSpec((1, 32), lambda g: (0, 0)),
                pl.BlockSpec((512, 64), lambda g: (0, 0)),
                pl.BlockSpec((1, 64), lambda g: (0, 0)),
                pl.BlockSpec((576, 64), lambda g: (0, 0)),
                pl.BlockSpec((1, 64), lambda g: (0, 0)),
            ],
            out_specs=pl.BlockSpec((1, bb, 5120), lambda g: (g, 0, 0)),
        ),
        compiler_params=pltpu.CompilerParams(
            dimension_semantics=("parallel",),
        ),
    )(xt, w1, conv1_b, w2, conv2_b, w3, conv3_b)
    flat = flat.reshape(B, 5120)

    return pl.pallas_call(
        _fc_head_kernel,
        out_shape=jax.ShapeDtypeStruct((B, _N_ACT), jnp.float32),
    )(flat, fc1_w, fc1_b, head_w, head_b)
